# baseline (speedup 1.0000x reference)
import jax
import jax.numpy as jnp
from jax.experimental import pallas as pl
from jax.experimental.pallas import tpu as pltpu
from jax.experimental.pallas import tpu_sc as plsc

K = 32
R = 128
E = R * K
IDX_MASK = 1023
TW = 128
WINDOW = 128


def _silu(x):
    return x * jax.nn.sigmoid(x)


def _roll3(x, shift):
    if shift == -1:
        return jnp.concatenate([x[:, 1:3], x[:, 0:1]], axis=1)
    return jnp.concatenate([x[:, 2:3], x[:, 0:2]], axis=1)


def _tile_k(x):
    return jnp.concatenate([x] * K, axis=0)


def _topk_block(feats_ref, coors_ref, coors_t_ref,
                aug_ref, idx_ref, dist_ref):
    n = feats_ref.shape[1]
    i0 = pl.program_id(1) * R

    coors_full = coors_ref[0]
    feats_blk = feats_ref[0, pl.ds(i0, R), :]
    coors_blk = coors_ref[0, pl.ds(i0, R), :]

    cxj = coors_t_ref[0, 0:1, :]
    cyj = coors_t_ref[0, 1:2, :]
    czj = coors_t_ref[0, 2:3, :]
    dx = coors_blk[:, 0:1] - cxj
    dy = coors_blk[:, 1:2] - cyj
    dz = coors_blk[:, 2:3] - czj
    d2 = dx * dx + dy * dy + dz * dz

    bits = jax.lax.bitcast_convert_type(d2, jnp.int32)
    jcol = jax.lax.broadcasted_iota(jnp.int32, (R, n), 1)
    arr = (bits & jnp.int32(~IDX_MASK)) | jcol
    maxval = jnp.int32(0x7FFFFFFF)
    cols = []
    for _ in range(K):
        m = jnp.min(arr, axis=1, keepdims=True)
        cols.append(m)
        arr = jnp.where(arr == m, maxval, arr)
    packed_flat = jnp.concatenate(cols, axis=0)
    idx_ref[0] = (packed_flat & IDX_MASK) + pl.program_id(0) * n
    dist_ref[0] = jax.lax.bitcast_convert_type(
        packed_flat & jnp.int32(~IDX_MASK), jnp.float32)

    mean_c = jnp.mean(coors_full, axis=0, keepdims=True)
    cnm_blk = coors_blk - mean_c
    pad = jnp.zeros((R, TW - 70), jnp.float32)
    aug_ref[0] = jnp.concatenate(
        [feats_blk, coors_blk, cnm_blk, pad], axis=1)


def _sc_gather(table, indices):
    out_rows = indices.shape[1]
    width = table.shape[1]
    mesh = plsc.VectorSubcoreMesh(core_axis_name="core",
                                  subcore_axis_name="subcore")

    @pl.kernel(out_type=jax.ShapeDtypeStruct((out_rows, width), table.dtype),
               mesh=mesh)
    def gk(tab_hbm, idx_hbm, o_hbm):
        def body(i_vmem, o_vmem):
            pltpu.sync_copy(tab_hbm.at[i_vmem.at[0]], o_vmem)

        pltpu.emit_pipeline(
            body,
            grid=(out_rows // WINDOW,),
            in_specs=[pl.BlockSpec((1, WINDOW), index_map=lambda i: (0, i))],
            out_specs=[pl.BlockSpec((WINDOW, width),
                                    index_map=lambda i: (i, 0))],
            core_axis_name=("core", "subcore"),
            dimension_semantics=(pltpu.PARALLEL,),
        )(idx_hbm, o_hbm)

    return gk(table, indices)


def _mlp_block(g_ref, dist_ref, aug_ref, feats_ref, coors_ref, pool_ref,
               W1a_ref, W1b_ref, w1c_ref, be1_ref, We2_ref, be2_ref,
               Wn1a_ref, Wn1b_ref, bn1_ref, Wn2_ref, bn2_ref,
               Wcx1_ref, bcx1_ref, Wcx2_ref, bcx2_ref,
               node_out_ref, coors_out_ref):
    feats_blk = feats_ref[0]
    coors_blk = coors_ref[0]
    dist_flat = dist_ref[0]
    Gj = g_ref[0]

    P_i = jnp.dot(feats_blk.astype(jnp.bfloat16), W1a_ref[...],
                  preferred_element_type=jnp.float32)
    fj = Gj[:, 0:64].astype(jnp.bfloat16)
    h = (_tile_k(P_i)
         + jnp.dot(fj, W1b_ref[...], preferred_element_type=jnp.float32)
         + dist_flat * w1c_ref[...]
         + be1_ref[...])
    h = _silu(h.astype(jnp.bfloat16))
    m_ij = _silu((jnp.dot(h, We2_ref[...],
                          preferred_element_type=jnp.float32)
                  + be2_ref[...]).astype(jnp.bfloat16))

    t12 = _silu((jnp.dot(m_ij, Wcx1_ref[...],
                         preferred_element_type=jnp.float32)
                 + bcx1_ref[...]).astype(jnp.bfloat16))
    cw2 = (jnp.dot(t12, Wcx2_ref[...],
                   preferred_element_type=jnp.float32)
           + bcx2_ref[...])
    cw = cw2[:, 0:1]
    cwx = cw2[:, 1:2]

    rel = _tile_k(coors_blk) - Gj[:, 64:67]
    ai = _tile_k(aug_ref[0][:, 67:70].astype(jnp.float32))
    bj = Gj[:, 67:70]
    cross = _roll3(ai, -1) * _roll3(bj, 1) - _roll3(ai, 1) * _roll3(bj, -1)
    contrib = cw * rel + cwx * cross

    pooled = jnp.dot(pool_ref[...],
                     jnp.concatenate([contrib.astype(jnp.bfloat16), m_ij],
                                     axis=1),
                     preferred_element_type=jnp.float32)
    csum = pooled[:, 0:3]
    m_i = pooled[:, 3:19]

    coors_out_ref[0] = csum + coors_blk

    nh = _silu((jnp.dot(feats_blk.astype(jnp.bfloat16), Wn1a_ref[...],
                        preferred_element_type=jnp.float32)
                + jnp.dot(m_i.astype(jnp.bfloat16), Wn1b_ref[...],
                          preferred_element_type=jnp.float32)
                + bn1_ref[...]).astype(jnp.bfloat16))
    node = (jnp.dot(nh, Wn2_ref[...],
                    preferred_element_type=jnp.float32)
            + bn2_ref[...] + feats_blk)
    node_out_ref[0] = node


@jax.jit
def kernel(feats, coors, W_e1, b_e1, W_e2, b_e2, W_n1, b_n1, W_n2, b_n2,
           W_c1, b_c1, W_c2, b_c2, W_x1, b_x1, W_x2, b_x2):
    b, n, d = feats.shape
    m_dim = W_e2.shape[1]
    nblk = n // R
    S = b * nblk
    NE = S * E

    coors_t = jnp.transpose(coors, (0, 2, 1))

    aug, idxg, distf = pl.pallas_call(
        _topk_block,
        grid=(b, nblk),
        in_specs=[
            pl.BlockSpec((1, n, d), lambda bi, ii: (bi, 0, 0)),
            pl.BlockSpec((1, n, 3), lambda bi, ii: (bi, 0, 0)),
            pl.BlockSpec((1, 3, n), lambda bi, ii: (bi, 0, 0)),
        ],
        out_specs=[
            pl.BlockSpec((1, R, TW), lambda bi, ii: (bi * nblk + ii, 0, 0)),
            pl.BlockSpec((1, E, 1), lambda bi, ii: (bi * nblk + ii, 0, 0)),
            pl.BlockSpec((1, E, 1), lambda bi, ii: (bi * nblk + ii, 0, 0)),
        ],
        out_shape=[
            jax.ShapeDtypeStruct((S, R, TW), jnp.float32),
            jax.ShapeDtypeStruct((S, E, 1), jnp.int32),
            jax.ShapeDtypeStruct((S, E, 1), jnp.float32),
        ],
    )(feats, coors, coors_t)

    table = aug.reshape(b * n, TW)
    indices = idxg.reshape(1, NE)
    gathered = _sc_gather(table, indices)
    gathered = gathered.reshape(S, E, TW)

    feats3 = feats.reshape(S, R, d)
    coors3 = coors.reshape(S, R, 3)

    pool = (jnp.arange(E, dtype=jnp.int32)[None, :] % R
            == jnp.arange(R, dtype=jnp.int32)[:, None]).astype(jnp.bfloat16)

    bf = jnp.bfloat16
    W1a = W_e1[:d].astype(bf)
    W1b = W_e1[d:2 * d].astype(bf)
    w1c = W_e1[2 * d:2 * d + 1]
    be1 = b_e1[None, :]
    We2 = W_e2.astype(bf)
    be2 = b_e2[None, :]
    Wn1a = W_n1[:d].astype(bf)
    Wn1b = W_n1[d:d + m_dim].astype(bf)
    bn1 = b_n1[None, :]
    Wn2 = W_n2.astype(bf)
    bn2 = b_n2[None, :]
    Wcx1 = jnp.concatenate([W_c1, W_x1], axis=1).astype(bf)
    bcx1 = jnp.concatenate([b_c1, b_x1])[None, :]
    zeros = jnp.zeros_like(W_c2)
    Wcx2 = jnp.concatenate(
        [jnp.concatenate([W_c2, zeros], axis=1),
         jnp.concatenate([zeros, W_x2], axis=1)], axis=0).astype(bf)
    bcx2 = jnp.concatenate([b_c2, b_x2])[None, :]

    full = lambda shp: pl.BlockSpec(shp, lambda s: (0,) * len(shp))
    step = lambda shp: pl.BlockSpec(shp, lambda s: (s,) + (0,) * (len(shp) - 1))

    node3, coors_out3 = pl.pallas_call(
        _mlp_block,
        grid=(S,),
        in_specs=[
            step((1, E, TW)),
            step((1, E, 1)),
            step((1, R, TW)),
            step((1, R, d)),
            step((1, R, 3)),
            full(pool.shape),
            full(W1a.shape), full(W1b.shape), full(w1c.shape), full(be1.shape),
            full(We2.shape), full(be2.shape),
            full(Wn1a.shape), full(Wn1b.shape), full(bn1.shape),
            full(Wn2.shape), full(bn2.shape),
            full(Wcx1.shape), full(bcx1.shape), full(Wcx2.shape),
            full(bcx2.shape),
        ],
        out_specs=[
            step((1, R, d)),
            step((1, R, 3)),
        ],
        out_shape=[
            jax.ShapeDtypeStruct((S, R, d), jnp.float32),
            jax.ShapeDtypeStruct((S, R, 3), jnp.float32),
        ],
    )(gathered, distf, aug, feats3, coors3, pool,
      W1a, W1b, w1c, be1, We2, be2, Wn1a, Wn1b, bn1, Wn2, bn2,
      Wcx1, bcx1, Wcx2, bcx2)

    return node3.reshape(b, n, d), coors_out3.reshape(b, n, 3)


# speedup vs baseline: 1.0036x; 1.0036x over previous
import jax
import jax.numpy as jnp
from jax.experimental import pallas as pl
from jax.experimental.pallas import tpu as pltpu
from jax.experimental.pallas import tpu_sc as plsc

K = 32
R = 128
E = R * K
IDX_MASK = 1023
TW = 128
WINDOW = 128


def _silu(x):
    return x * jax.nn.sigmoid(x)


def _roll3(x, shift):
    if shift == -1:
        return jnp.concatenate([x[:, 1:3], x[:, 0:1]], axis=1)
    return jnp.concatenate([x[:, 2:3], x[:, 0:2]], axis=1)


def _tile_k(x):
    return jnp.concatenate([x] * K, axis=0)


def _topk_block(feats_ref, coors_ref, coors_t_ref,
                aug_ref, idx_ref, dist_ref):
    n = feats_ref.shape[1]
    i0 = pl.program_id(1) * R

    coors_full = coors_ref[0]
    feats_blk = feats_ref[0, pl.ds(i0, R), :]
    coors_blk = coors_ref[0, pl.ds(i0, R), :]

    cxj = coors_t_ref[0, 0:1, :]
    cyj = coors_t_ref[0, 1:2, :]
    czj = coors_t_ref[0, 2:3, :]
    dx = coors_blk[:, 0:1] - cxj
    dy = coors_blk[:, 1:2] - cyj
    dz = coors_blk[:, 2:3] - czj
    d2 = dx * dx + dy * dy + dz * dz

    bits = jax.lax.bitcast_convert_type(d2, jnp.int32)
    jcol = jax.lax.broadcasted_iota(jnp.int32, (R, n), 1)
    arr = (bits & jnp.int32(~IDX_MASK)) | jcol
    maxval = jnp.int32(0x7FFFFFFF)
    cols = []
    for _ in range(K):
        m = jnp.min(arr, axis=1, keepdims=True)
        cols.append(m)
        arr = jnp.where(arr == m, maxval, arr)
    packed_flat = jnp.concatenate(cols, axis=0)
    idx_ref[0] = packed_flat & IDX_MASK
    dist_ref[0] = jax.lax.bitcast_convert_type(
        packed_flat & jnp.int32(~IDX_MASK), jnp.float32)

    mean_c = jnp.mean(coors_full, axis=0, keepdims=True)
    cnm_blk = coors_blk - mean_c
    pad = jnp.zeros((R, TW - 70), jnp.float32)
    aug_ref[0] = jnp.concatenate(
        [feats_blk, coors_blk, cnm_blk, pad], axis=1)


def _sc_gather(table, indices):
    out_rows = indices.shape[1]
    width = table.shape[1]
    mesh = plsc.VectorSubcoreMesh(core_axis_name="core",
                                  subcore_axis_name="subcore")

    @pl.kernel(out_type=jax.ShapeDtypeStruct((out_rows, width), table.dtype),
               mesh=mesh)
    def gk(tab_hbm, idx_hbm, o_hbm):
        def body(i_vmem, o_vmem):
            pltpu.sync_copy(tab_hbm.at[i_vmem.at[0]], o_vmem)

        pltpu.emit_pipeline(
            body,
            grid=(out_rows // WINDOW,),
            in_specs=[pl.BlockSpec((1, WINDOW), index_map=lambda i: (0, i))],
            out_specs=[pl.BlockSpec((WINDOW, width),
                                    index_map=lambda i: (i, 0))],
            core_axis_name=("core", "subcore"),
            dimension_semantics=(pltpu.PARALLEL,),
        )(idx_hbm, o_hbm)

    return gk(table, indices)


def _mlp_block(g_ref, dist_ref, aug_ref, feats_ref, coors_ref, pool_ref,
               W1a_ref, W1b_ref, w1c_ref, be1_ref, We2_ref, be2_ref,
               Wn1a_ref, Wn1b_ref, bn1_ref, Wn2_ref, bn2_ref,
               Wcx1_ref, bcx1_ref, Wcx2_ref, bcx2_ref,
               node_out_ref, coors_out_ref):
    feats_blk = feats_ref[0]
    coors_blk = coors_ref[0]
    dist_flat = dist_ref[0]
    Gj = g_ref[0]

    P_i = jnp.dot(feats_blk.astype(jnp.bfloat16), W1a_ref[...],
                  preferred_element_type=jnp.float32)
    fj = Gj[:, 0:64].astype(jnp.bfloat16)
    h = (_tile_k(P_i)
         + jnp.dot(fj, W1b_ref[...], preferred_element_type=jnp.float32)
         + dist_flat * w1c_ref[...]
         + be1_ref[...])
    h = _silu(h.astype(jnp.bfloat16))
    m_ij = _silu((jnp.dot(h, We2_ref[...],
                          preferred_element_type=jnp.float32)
                  + be2_ref[...]).astype(jnp.bfloat16))

    t12 = _silu((jnp.dot(m_ij, Wcx1_ref[...],
                         preferred_element_type=jnp.float32)
                 + bcx1_ref[...]).astype(jnp.bfloat16))
    cw2 = (jnp.dot(t12, Wcx2_ref[...],
                   preferred_element_type=jnp.float32)
           + bcx2_ref[...])
    cw = cw2[:, 0:1]
    cwx = cw2[:, 1:2]

    rel = _tile_k(coors_blk) - Gj[:, 64:67]
    ai = _tile_k(aug_ref[0][:, 67:70].astype(jnp.float32))
    bj = Gj[:, 67:70]
    cross = _roll3(ai, -1) * _roll3(bj, 1) - _roll3(ai, 1) * _roll3(bj, -1)
    contrib = cw * rel + cwx * cross

    pooled = jnp.dot(pool_ref[...],
                     jnp.concatenate([contrib.astype(jnp.bfloat16), m_ij],
                                     axis=1),
                     preferred_element_type=jnp.float32)
    csum = pooled[:, 0:3]
    m_i = pooled[:, 3:19]

    coors_out_ref[0] = csum + coors_blk

    nh = _silu((jnp.dot(feats_blk.astype(jnp.bfloat16), Wn1a_ref[...],
                        preferred_element_type=jnp.float32)
                + jnp.dot(m_i.astype(jnp.bfloat16), Wn1b_ref[...],
                          preferred_element_type=jnp.float32)
                + bn1_ref[...]).astype(jnp.bfloat16))
    node = (jnp.dot(nh, Wn2_ref[...],
                    preferred_element_type=jnp.float32)
            + bn2_ref[...] + feats_blk)
    node_out_ref[0] = node


@jax.jit
def kernel(feats, coors, W_e1, b_e1, W_e2, b_e2, W_n1, b_n1, W_n2, b_n2,
           W_c1, b_c1, W_c2, b_c2, W_x1, b_x1, W_x2, b_x2):
    b, n, d = feats.shape
    m_dim = W_e2.shape[1]
    nblk = n // R

    coors_t = jnp.transpose(coors, (0, 2, 1))

    pool = (jnp.arange(E, dtype=jnp.int32)[None, :] % R
            == jnp.arange(R, dtype=jnp.int32)[:, None]).astype(jnp.bfloat16)

    bf = jnp.bfloat16
    W1a = W_e1[:d].astype(bf)
    W1b = W_e1[d:2 * d].astype(bf)
    w1c = W_e1[2 * d:2 * d + 1]
    be1 = b_e1[None, :]
    We2 = W_e2.astype(bf)
    be2 = b_e2[None, :]
    Wn1a = W_n1[:d].astype(bf)
    Wn1b = W_n1[d:d + m_dim].astype(bf)
    bn1 = b_n1[None, :]
    Wn2 = W_n2.astype(bf)
    bn2 = b_n2[None, :]
    Wcx1 = jnp.concatenate([W_c1, W_x1], axis=1).astype(bf)
    bcx1 = jnp.concatenate([b_c1, b_x1])[None, :]
    zeros = jnp.zeros_like(W_c2)
    Wcx2 = jnp.concatenate(
        [jnp.concatenate([W_c2, zeros], axis=1),
         jnp.concatenate([zeros, W_x2], axis=1)], axis=0).astype(bf)
    bcx2 = jnp.concatenate([b_c2, b_x2])[None, :]

    full = lambda shp: pl.BlockSpec(shp, lambda s: (0,) * len(shp))
    step = lambda shp: pl.BlockSpec(shp, lambda s: (s,) + (0,) * (len(shp) - 1))
    fullA = lambda shp: pl.BlockSpec(shp, lambda bi, ii: (0,) * len(shp))
    stepA = lambda shp: pl.BlockSpec(
        shp, lambda bi, ii: (ii,) + (0,) * (len(shp) - 1))

    node_parts, coors_parts = [], []
    for bi in range(b):
        aug, idxg, distf = pl.pallas_call(
            _topk_block,
            grid=(1, nblk),
            in_specs=[
                fullA((1, n, d)),
                fullA((1, n, 3)),
                fullA((1, 3, n)),
            ],
            out_specs=[
                stepA((1, R, TW)),
                stepA((1, E, 1)),
                stepA((1, E, 1)),
            ],
            out_shape=[
                jax.ShapeDtypeStruct((nblk, R, TW), jnp.float32),
                jax.ShapeDtypeStruct((nblk, E, 1), jnp.int32),
                jax.ShapeDtypeStruct((nblk, E, 1), jnp.float32),
            ],
        )(feats[bi:bi + 1], coors[bi:bi + 1], coors_t[bi:bi + 1])

        table = aug.reshape(n, TW)
        indices = idxg.reshape(1, nblk * E)
        gathered = _sc_gather(table, indices).reshape(nblk, E, TW)

        node3, coors3 = pl.pallas_call(
            _mlp_block,
            grid=(nblk,),
            in_specs=[
                step((1, E, TW)),
                step((1, E, 1)),
                step((1, R, TW)),
                step((1, R, d)),
                step((1, R, 3)),
                full(pool.shape),
                full(W1a.shape), full(W1b.shape), full(w1c.shape),
                full(be1.shape), full(We2.shape), full(be2.shape),
                full(Wn1a.shape), full(Wn1b.shape), full(bn1.shape),
                full(Wn2.shape), full(bn2.shape),
                full(Wcx1.shape), full(bcx1.shape), full(Wcx2.shape),
                full(bcx2.shape),
            ],
            out_specs=[
                step((1, R, d)),
                step((1, R, 3)),
            ],
            out_shape=[
                jax.ShapeDtypeStruct((nblk, R, d), jnp.float32),
                jax.ShapeDtypeStruct((nblk, R, 3), jnp.float32),
            ],
        )(gathered, distf, aug, feats[bi].reshape(nblk, R, d),
          coors[bi].reshape(nblk, R, 3), pool,
          W1a, W1b, w1c, be1, We2, be2, Wn1a, Wn1b, bn1, Wn2, bn2,
          Wcx1, bcx1, Wcx2, bcx2)
        node_parts.append(node3.reshape(1, n, d))
        coors_parts.append(coors3.reshape(1, n, 3))

    return (jnp.concatenate(node_parts, axis=0),
            jnp.concatenate(coors_parts, axis=0))


# speedup vs baseline: 1.2089x; 1.2046x over previous
import jax
import jax.numpy as jnp
from jax.experimental import pallas as pl
from jax.experimental.pallas import tpu as pltpu
from jax.experimental.pallas import tpu_sc as plsc

K = 32
R = 128
E = R * K
IDX_MASK = 1023
TW = 128
WINDOW = 128


def _silu(x):
    half = jnp.asarray(0.5, x.dtype)
    one = jnp.asarray(1.0, x.dtype)
    return half * x * (one + jnp.tanh(half * x))


def _roll3(x, shift):
    if shift == -1:
        return jnp.concatenate([x[:, 1:3], x[:, 0:1]], axis=1)
    return jnp.concatenate([x[:, 2:3], x[:, 0:2]], axis=1)


def _tile_k(x):
    return jnp.concatenate([x] * K, axis=0)


def _topk_block(feats_ref, coors_ref, coors_t_ref,
                aug_ref, idx_ref, dist_ref):
    n = feats_ref.shape[1]
    i0 = pl.program_id(1) * R

    coors_full = coors_ref[0]
    feats_blk = feats_ref[0, pl.ds(i0, R), :]
    coors_blk = coors_ref[0, pl.ds(i0, R), :]

    cxi = coors_t_ref[0, 0:1, pl.ds(i0, R)]
    cyi = coors_t_ref[0, 1:2, pl.ds(i0, R)]
    czi = coors_t_ref[0, 2:3, pl.ds(i0, R)]
    dx = coors_full[:, 0:1] - cxi
    dy = coors_full[:, 1:2] - cyi
    dz = coors_full[:, 2:3] - czi
    d2 = dx * dx + dy * dy + dz * dz

    bits = jax.lax.bitcast_convert_type(d2, jnp.int32)
    jrow = jax.lax.broadcasted_iota(jnp.int32, (n, R), 0)
    arr = (bits & jnp.int32(~IDX_MASK)) | jrow
    maxval = jnp.int32(0x7FFFFFFF)
    rows = []
    for _ in range(K):
        m = jnp.min(arr, axis=0, keepdims=True)
        rows.append(m)
        arr = jnp.where(arr == m, maxval, arr)
    packed_kr = jnp.concatenate(rows, axis=0)
    idx_ref[0] = packed_kr & IDX_MASK
    dist_ref[0] = jax.lax.bitcast_convert_type(
        packed_kr & jnp.int32(~IDX_MASK), jnp.float32)

    mean_c = jnp.mean(coors_full, axis=0, keepdims=True)
    cnm_blk = coors_blk - mean_c
    pad = jnp.zeros((R, TW - 70), jnp.float32)
    aug_ref[0] = jnp.concatenate(
        [feats_blk, coors_blk, cnm_blk, pad], axis=1)


def _sc_gather(table, indices):
    out_rows = indices.shape[1]
    width = table.shape[1]
    mesh = plsc.VectorSubcoreMesh(core_axis_name="core",
                                  subcore_axis_name="subcore")

    @pl.kernel(out_type=jax.ShapeDtypeStruct((out_rows, width), table.dtype),
               mesh=mesh)
    def gk(tab_hbm, idx_hbm, o_hbm):
        def body(i_vmem, o_vmem):
            pltpu.sync_copy(tab_hbm.at[i_vmem.at[0]], o_vmem)

        pltpu.emit_pipeline(
            body,
            grid=(out_rows // WINDOW,),
            in_specs=[pl.BlockSpec((1, WINDOW), index_map=lambda i: (0, i))],
            out_specs=[pl.BlockSpec((WINDOW, width),
                                    index_map=lambda i: (i, 0))],
            core_axis_name=("core", "subcore"),
            dimension_semantics=(pltpu.PARALLEL,),
        )(idx_hbm, o_hbm)

    return gk(table, indices)


def _mlp_block(g_ref, dist_ref, aug_ref, feats_ref, coors_ref, pool_ref,
               W1a_ref, W1b_ref, w1c_ref, be1_ref, We2_ref, be2_ref,
               Wn1a_ref, Wn1b_ref, bn1_ref, Wn2_ref, bn2_ref,
               Wcx1_ref, bcx1_ref, Wcx2_ref, bcx2_ref,
               node_out_ref, coors_out_ref):
    feats_blk = feats_ref[0]
    coors_blk = coors_ref[0]
    dist_flat = dist_ref[0]

    P_i = jnp.dot(feats_blk.astype(jnp.bfloat16), W1a_ref[...],
                  preferred_element_type=jnp.float32)
    fj = g_ref[0, :, 0:64].astype(jnp.bfloat16)
    h = (_tile_k(P_i)
         + jnp.dot(fj, W1b_ref[...], preferred_element_type=jnp.float32)
         + dist_flat * w1c_ref[...]
         + be1_ref[...])
    h = _silu(h.astype(jnp.bfloat16))
    m_ij = _silu((jnp.dot(h, We2_ref[...],
                          preferred_element_type=jnp.float32)
                  + be2_ref[...]).astype(jnp.bfloat16))

    t12 = _silu((jnp.dot(m_ij, Wcx1_ref[...],
                         preferred_element_type=jnp.float32)
                 + bcx1_ref[...]).astype(jnp.bfloat16))
    cw2 = (jnp.dot(t12, Wcx2_ref[...],
                   preferred_element_type=jnp.float32)
           + bcx2_ref[...])
    cw = cw2[:, 0:1]
    cwx = cw2[:, 1:2]

    rel = _tile_k(coors_blk) - g_ref[0, :, 64:67]
    ai = _tile_k(aug_ref[0, :, 67:70])
    bj = g_ref[0, :, 67:70]
    cross = _roll3(ai, -1) * _roll3(bj, 1) - _roll3(ai, 1) * _roll3(bj, -1)
    contrib = cw * rel + cwx * cross

    pooled = jnp.dot(pool_ref[...],
                     jnp.concatenate([contrib.astype(jnp.bfloat16), m_ij],
                                     axis=1),
                     preferred_element_type=jnp.float32)
    csum = pooled[:, 0:3]
    m_i = pooled[:, 3:19]

    coors_out_ref[0] = csum + coors_blk

    nh = _silu((jnp.dot(feats_blk.astype(jnp.bfloat16), Wn1a_ref[...],
                        preferred_element_type=jnp.float32)
                + jnp.dot(m_i.astype(jnp.bfloat16), Wn1b_ref[...],
                          preferred_element_type=jnp.float32)
                + bn1_ref[...]).astype(jnp.bfloat16))
    node = (jnp.dot(nh, Wn2_ref[...],
                    preferred_element_type=jnp.float32)
            + bn2_ref[...] + feats_blk)
    node_out_ref[0] = node


@jax.jit
def kernel(feats, coors, W_e1, b_e1, W_e2, b_e2, W_n1, b_n1, W_n2, b_n2,
           W_c1, b_c1, W_c2, b_c2, W_x1, b_x1, W_x2, b_x2):
    b, n, d = feats.shape
    m_dim = W_e2.shape[1]
    nblk = n // R

    coors_t = jnp.transpose(coors, (0, 2, 1))

    pool = (jnp.arange(E, dtype=jnp.int32)[None, :] % R
            == jnp.arange(R, dtype=jnp.int32)[:, None]).astype(jnp.bfloat16)

    bf = jnp.bfloat16
    W1a = W_e1[:d].astype(bf)
    W1b = W_e1[d:2 * d].astype(bf)
    w1c = W_e1[2 * d:2 * d + 1]
    be1 = b_e1[None, :]
    We2 = W_e2.astype(bf)
    be2 = b_e2[None, :]
    Wn1a = W_n1[:d].astype(bf)
    Wn1b = W_n1[d:d + m_dim].astype(bf)
    bn1 = b_n1[None, :]
    Wn2 = W_n2.astype(bf)
    bn2 = b_n2[None, :]
    Wcx1 = jnp.concatenate([W_c1, W_x1], axis=1).astype(bf)
    bcx1 = jnp.concatenate([b_c1, b_x1])[None, :]
    zeros = jnp.zeros_like(W_c2)
    Wcx2 = jnp.concatenate(
        [jnp.concatenate([W_c2, zeros], axis=1),
         jnp.concatenate([zeros, W_x2], axis=1)], axis=0).astype(bf)
    bcx2 = jnp.concatenate([b_c2, b_x2])[None, :]

    full = lambda shp: pl.BlockSpec(shp, lambda s: (0,) * len(shp))
    step = lambda shp: pl.BlockSpec(shp, lambda s: (s,) + (0,) * (len(shp) - 1))
    fullA = lambda shp: pl.BlockSpec(shp, lambda bi, ii: (0,) * len(shp))
    stepA = lambda shp: pl.BlockSpec(
        shp, lambda bi, ii: (ii,) + (0,) * (len(shp) - 1))

    node_parts, coors_parts = [], []
    for bi in range(b):
        aug, idxg, distf = pl.pallas_call(
            _topk_block,
            grid=(1, nblk),
            in_specs=[
                fullA((1, n, d)),
                fullA((1, n, 3)),
                fullA((1, 3, n)),
            ],
            out_specs=[
                stepA((1, R, TW)),
                stepA((1, K, R)),
                stepA((1, K, R)),
            ],
            out_shape=[
                jax.ShapeDtypeStruct((nblk, R, TW), jnp.float32),
                jax.ShapeDtypeStruct((nblk, K, R), jnp.int32),
                jax.ShapeDtypeStruct((nblk, K, R), jnp.float32),
            ],
        )(feats[bi:bi + 1], coors[bi:bi + 1], coors_t[bi:bi + 1])

        table = aug.reshape(n, TW)
        indices = idxg.reshape(1, nblk * E)
        distf = distf.reshape(nblk, E, 1)
        gathered = _sc_gather(table, indices).reshape(nblk, E, TW)

        node3, coors3 = pl.pallas_call(
            _mlp_block,
            grid=(nblk,),
            in_specs=[
                step((1, E, TW)),
                step((1, E, 1)),
                step((1, R, TW)),
                step((1, R, d)),
                step((1, R, 3)),
                full(pool.shape),
                full(W1a.shape), full(W1b.shape), full(w1c.shape),
                full(be1.shape), full(We2.shape), full(be2.shape),
                full(Wn1a.shape), full(Wn1b.shape), full(bn1.shape),
                full(Wn2.shape), full(bn2.shape),
                full(Wcx1.shape), full(bcx1.shape), full(Wcx2.shape),
                full(bcx2.shape),
            ],
            out_specs=[
                step((1, R, d)),
                step((1, R, 3)),
            ],
            out_shape=[
                jax.ShapeDtypeStruct((nblk, R, d), jnp.float32),
                jax.ShapeDtypeStruct((nblk, R, 3), jnp.float32),
            ],
        )(gathered, distf, aug, feats[bi].reshape(nblk, R, d),
          coors[bi].reshape(nblk, R, 3), pool,
          W1a, W1b, w1c, be1, We2, be2, Wn1a, Wn1b, bn1, Wn2, bn2,
          Wcx1, bcx1, Wcx2, bcx2)
        node_parts.append(node3.reshape(1, n, d))
        coors_parts.append(coors3.reshape(1, n, 3))

    return (jnp.concatenate(node_parts, axis=0),
            jnp.concatenate(coors_parts, axis=0))


# speedup vs baseline: 1.2439x; 1.0290x over previous
import jax
import jax.numpy as jnp
from jax.experimental import pallas as pl
from jax.experimental.pallas import tpu as pltpu
from jax.experimental.pallas import tpu_sc as plsc

K = 32
R = 128
E = R * K
IDX_MASK = 1023
TW = 128
WINDOW = 128


def _silu(x):
    half = jnp.asarray(0.5, x.dtype)
    one = jnp.asarray(1.0, x.dtype)
    return half * x * (one + jnp.tanh(half * x))


def _roll3(x, shift):
    if shift == -1:
        return jnp.concatenate([x[:, 1:3], x[:, 0:1]], axis=1)
    return jnp.concatenate([x[:, 2:3], x[:, 0:2]], axis=1)


def _tile_k(x):
    return jnp.concatenate([x] * K, axis=0)


def _topk_block(feats_ref, coors_ref, coors_t_ref,
                aug_ref, idx_ref, dist_ref):
    n = feats_ref.shape[1]
    i0 = pl.program_id(1) * R

    coors_full = coors_ref[0]
    feats_blk = feats_ref[0, pl.ds(i0, R), :]
    coors_blk = coors_ref[0, pl.ds(i0, R), :]

    cxi = coors_t_ref[0, 0:1, pl.ds(i0, R)]
    cyi = coors_t_ref[0, 1:2, pl.ds(i0, R)]
    czi = coors_t_ref[0, 2:3, pl.ds(i0, R)]
    dx = coors_full[:, 0:1] - cxi
    dy = coors_full[:, 1:2] - cyi
    dz = coors_full[:, 2:3] - czi
    d2 = dx * dx + dy * dy + dz * dz

    bits = jax.lax.bitcast_convert_type(d2, jnp.int32)
    jrow = jax.lax.broadcasted_iota(jnp.int32, (n, R), 0)
    arr = (bits & jnp.int32(~IDX_MASK)) | jrow
    maxval = jnp.int32(0x7FFFFFFF)
    rows = []
    for _ in range(K):
        m = jnp.min(arr, axis=0, keepdims=True)
        rows.append(m)
        arr = jnp.where(arr == m, maxval, arr)
    packed_kr = jnp.concatenate(rows, axis=0)
    idx_ref[0] = packed_kr & IDX_MASK
    dist_ref[0] = jax.lax.bitcast_convert_type(
        packed_kr & jnp.int32(~IDX_MASK), jnp.float32)

    mean_c = jnp.mean(coors_full, axis=0, keepdims=True)
    cnm_blk = coors_blk - mean_c
    pad = jnp.zeros((R, TW - 70), jnp.float32)
    aug_ref[0] = jnp.concatenate(
        [feats_blk, coors_blk, cnm_blk, pad], axis=1)


def _sc_gather(table, indices):
    out_rows = indices.shape[1]
    width = table.shape[1]
    mesh = plsc.VectorSubcoreMesh(core_axis_name="core",
                                  subcore_axis_name="subcore")

    @pl.kernel(out_type=jax.ShapeDtypeStruct((out_rows, width), table.dtype),
               mesh=mesh)
    def gk(tab_hbm, idx_hbm, o_hbm):
        def body(i_vmem, o_vmem):
            pltpu.sync_copy(tab_hbm.at[i_vmem.at[0]], o_vmem)

        pltpu.emit_pipeline(
            body,
            grid=(out_rows // WINDOW,),
            in_specs=[pl.BlockSpec((1, WINDOW), index_map=lambda i: (0, i))],
            out_specs=[pl.BlockSpec((WINDOW, width),
                                    index_map=lambda i: (i, 0))],
            core_axis_name=("core", "subcore"),
            dimension_semantics=(pltpu.PARALLEL,),
        )(idx_hbm, o_hbm)

    return gk(table, indices)


def _mlp_block(g_ref, dist_ref, aug_ref, feats_ref, coors_ref, pool_ref,
               W1a_ref, W1b_ref, w1c_ref, be1_ref, We2_ref, be2_ref,
               Wn1a_ref, Wn1b_ref, bn1_ref, Wn2_ref, bn2_ref,
               Wcx1_ref, bcx1_ref, Wcx2_ref, bcx2_ref,
               node_out_ref, coors_out_ref):
    feats_blk = feats_ref[0]
    coors_blk = coors_ref[0]
    dist_flat = dist_ref[0]

    P_i = (jnp.dot(feats_blk.astype(jnp.bfloat16), W1a_ref[...],
                   preferred_element_type=jnp.float32)
           + be1_ref[...]).astype(jnp.bfloat16)
    fj = g_ref[0, :, 0:64].astype(jnp.bfloat16)
    h = (_tile_k(P_i)
         + jnp.dot(fj, W1b_ref[...],
                   preferred_element_type=jnp.float32).astype(jnp.bfloat16)
         + dist_flat.astype(jnp.bfloat16) * w1c_ref[...])
    h = _silu(h)
    m_ij = _silu((jnp.dot(h, We2_ref[...],
                          preferred_element_type=jnp.float32)
                  + be2_ref[...]).astype(jnp.bfloat16))

    t12 = _silu((jnp.dot(m_ij, Wcx1_ref[...],
                         preferred_element_type=jnp.float32)
                 + bcx1_ref[...]).astype(jnp.bfloat16))
    cw2 = (jnp.dot(t12, Wcx2_ref[...],
                   preferred_element_type=jnp.float32)
           + bcx2_ref[...])
    cw = cw2[:, 0:1]
    cwx = cw2[:, 1:2]

    rel = _tile_k(coors_blk) - g_ref[0, :, 64:67]
    ai = _tile_k(aug_ref[0, :, 67:70])
    bj = g_ref[0, :, 67:70]
    cross = _roll3(ai, -1) * _roll3(bj, 1) - _roll3(ai, 1) * _roll3(bj, -1)
    contrib = cw * rel + cwx * cross

    pooled = jnp.dot(pool_ref[...],
                     jnp.concatenate([contrib.astype(jnp.bfloat16), m_ij],
                                     axis=1),
                     preferred_element_type=jnp.float32)
    csum = pooled[:, 0:3]
    m_i = pooled[:, 3:19]

    coors_out_ref[0] = csum + coors_blk

    nh = _silu((jnp.dot(feats_blk.astype(jnp.bfloat16), Wn1a_ref[...],
                        preferred_element_type=jnp.float32)
                + jnp.dot(m_i.astype(jnp.bfloat16), Wn1b_ref[...],
                          preferred_element_type=jnp.float32)
                + bn1_ref[...]).astype(jnp.bfloat16))
    node = (jnp.dot(nh, Wn2_ref[...],
                    preferred_element_type=jnp.float32)
            + bn2_ref[...] + feats_blk)
    node_out_ref[0] = node


@jax.jit
def kernel(feats, coors, W_e1, b_e1, W_e2, b_e2, W_n1, b_n1, W_n2, b_n2,
           W_c1, b_c1, W_c2, b_c2, W_x1, b_x1, W_x2, b_x2):
    b, n, d = feats.shape
    m_dim = W_e2.shape[1]
    nblk = n // R

    coors_t = jnp.transpose(coors, (0, 2, 1))

    pool = (jnp.arange(E, dtype=jnp.int32)[None, :] % R
            == jnp.arange(R, dtype=jnp.int32)[:, None]).astype(jnp.bfloat16)

    bf = jnp.bfloat16
    W1a = W_e1[:d].astype(bf)
    W1b = W_e1[d:2 * d].astype(bf)
    w1c = W_e1[2 * d:2 * d + 1].astype(bf)
    be1 = b_e1[None, :]
    We2 = W_e2.astype(bf)
    be2 = b_e2[None, :]
    Wn1a = W_n1[:d].astype(bf)
    Wn1b = W_n1[d:d + m_dim].astype(bf)
    bn1 = b_n1[None, :]
    Wn2 = W_n2.astype(bf)
    bn2 = b_n2[None, :]
    Wcx1 = jnp.concatenate([W_c1, W_x1], axis=1).astype(bf)
    bcx1 = jnp.concatenate([b_c1, b_x1])[None, :]
    zeros = jnp.zeros_like(W_c2)
    Wcx2 = jnp.concatenate(
        [jnp.concatenate([W_c2, zeros], axis=1),
         jnp.concatenate([zeros, W_x2], axis=1)], axis=0).astype(bf)
    bcx2 = jnp.concatenate([b_c2, b_x2])[None, :]

    full = lambda shp: pl.BlockSpec(shp, lambda s: (0,) * len(shp))
    step = lambda shp: pl.BlockSpec(shp, lambda s: (s,) + (0,) * (len(shp) - 1))
    fullA = lambda shp: pl.BlockSpec(shp, lambda bi, ii: (0,) * len(shp))
    stepA = lambda shp: pl.BlockSpec(
        shp, lambda bi, ii: (ii,) + (0,) * (len(shp) - 1))

    node_parts, coors_parts = [], []
    for bi in range(b):
        aug, idxg, distf = pl.pallas_call(
            _topk_block,
            grid=(1, nblk),
            in_specs=[
                fullA((1, n, d)),
                fullA((1, n, 3)),
                fullA((1, 3, n)),
            ],
            out_specs=[
                stepA((1, R, TW)),
                stepA((1, K, R)),
                stepA((1, K, R)),
            ],
            out_shape=[
                jax.ShapeDtypeStruct((nblk, R, TW), jnp.float32),
                jax.ShapeDtypeStruct((nblk, K, R), jnp.int32),
                jax.ShapeDtypeStruct((nblk, K, R), jnp.float32),
            ],
        )(feats[bi:bi + 1], coors[bi:bi + 1], coors_t[bi:bi + 1])

        table = aug.reshape(n, TW)
        indices = idxg.reshape(1, nblk * E)
        distf = distf.reshape(nblk, E, 1)
        gathered = _sc_gather(table, indices).reshape(nblk, E, TW)

        node3, coors3 = pl.pallas_call(
            _mlp_block,
            grid=(nblk,),
            in_specs=[
                step((1, E, TW)),
                step((1, E, 1)),
                step((1, R, TW)),
                step((1, R, d)),
                step((1, R, 3)),
                full(pool.shape),
                full(W1a.shape), full(W1b.shape), full(w1c.shape),
                full(be1.shape), full(We2.shape), full(be2.shape),
                full(Wn1a.shape), full(Wn1b.shape), full(bn1.shape),
                full(Wn2.shape), full(bn2.shape),
                full(Wcx1.shape), full(bcx1.shape), full(Wcx2.shape),
                full(bcx2.shape),
            ],
            out_specs=[
                step((1, R, d)),
                step((1, R, 3)),
            ],
            out_shape=[
                jax.ShapeDtypeStruct((nblk, R, d), jnp.float32),
                jax.ShapeDtypeStruct((nblk, R, 3), jnp.float32),
            ],
        )(gathered, distf, aug, feats[bi].reshape(nblk, R, d),
          coors[bi].reshape(nblk, R, 3), pool,
          W1a, W1b, w1c, be1, We2, be2, Wn1a, Wn1b, bn1, Wn2, bn2,
          Wcx1, bcx1, Wcx2, bcx2)
        node_parts.append(node3.reshape(1, n, d))
        coors_parts.append(coors3.reshape(1, n, 3))

    return (jnp.concatenate(node_parts, axis=0),
            jnp.concatenate(coors_parts, axis=0))


# speedup vs baseline: 1.2993x; 1.0445x over previous
import jax
import jax.numpy as jnp
from jax.experimental import pallas as pl
from jax.experimental.pallas import tpu as pltpu
from jax.experimental.pallas import tpu_sc as plsc

K = 32
R = 128
E = R * K
IDX_MASK = 1023
TW = 128
WINDOW = 128


def _silu(x):
    half = jnp.asarray(0.5, x.dtype)
    one = jnp.asarray(1.0, x.dtype)
    return half * x * (one + jnp.tanh(half * x))


def _roll3(x, shift):
    if shift == -1:
        return jnp.concatenate([x[:, 1:3], x[:, 0:1]], axis=1)
    return jnp.concatenate([x[:, 2:3], x[:, 0:2]], axis=1)


def _tile_k(x):
    return jnp.concatenate([x] * K, axis=0)


def _topk_block(feats_ref, coors_ref, coors_t_ref,
                aug_ref, idx_ref, dist_ref):
    n = feats_ref.shape[1]
    i0 = pl.program_id(1) * R

    coors_full = coors_ref[0]
    feats_blk = feats_ref[0, pl.ds(i0, R), :]
    coors_blk = coors_ref[0, pl.ds(i0, R), :]

    cxi = coors_t_ref[0, 0:1, pl.ds(i0, R)]
    cyi = coors_t_ref[0, 1:2, pl.ds(i0, R)]
    czi = coors_t_ref[0, 2:3, pl.ds(i0, R)]
    dx = coors_full[:, 0:1] - cxi
    dy = coors_full[:, 1:2] - cyi
    dz = coors_full[:, 2:3] - czi
    d2 = dx * dx + dy * dy + dz * dz

    bits = jax.lax.bitcast_convert_type(d2, jnp.int32)
    jrow = jax.lax.broadcasted_iota(jnp.int32, (n, R), 0)
    arr = (bits & jnp.int32(~IDX_MASK)) | jrow
    maxval = jnp.int32(0x7FFFFFFF)
    rows = []
    for _ in range(K):
        m = jnp.min(arr, axis=0, keepdims=True)
        rows.append(m)
        arr = jnp.where(arr == m, maxval, arr)
    packed_kr = jnp.concatenate(rows, axis=0)
    idx_ref[0] = (packed_kr & IDX_MASK) + pl.program_id(0) * n
    dist_ref[0] = jax.lax.bitcast_convert_type(
        packed_kr & jnp.int32(~IDX_MASK), jnp.float32)

    mean_c = jnp.mean(coors_full, axis=0, keepdims=True)
    cnm_blk = coors_blk - mean_c
    pad = jnp.zeros((R, TW - 70), jnp.float32)
    aug_ref[0] = jnp.concatenate(
        [feats_blk, coors_blk, cnm_blk, pad], axis=1)


def _sc_gather(table, indices):
    out_rows = indices.shape[1]
    width = table.shape[1]
    mesh = plsc.VectorSubcoreMesh(core_axis_name="core",
                                  subcore_axis_name="subcore")

    @pl.kernel(out_type=jax.ShapeDtypeStruct((out_rows, width), table.dtype),
               mesh=mesh)
    def gk(tab_hbm, idx_hbm, o_hbm):
        def body(i_vmem, o_vmem):
            pltpu.sync_copy(tab_hbm.at[i_vmem.at[0]], o_vmem)

        pltpu.emit_pipeline(
            body,
            grid=(out_rows // WINDOW,),
            in_specs=[pl.BlockSpec((1, WINDOW), index_map=lambda i: (0, i))],
            out_specs=[pl.BlockSpec((WINDOW, width),
                                    index_map=lambda i: (i, 0))],
            core_axis_name=("core", "subcore"),
            dimension_semantics=(pltpu.PARALLEL,),
        )(idx_hbm, o_hbm)

    return gk(table, indices)


def _mlp_block(g_ref, dist_ref, aug_ref, feats_ref, coors_ref, pool_ref,
               W1a_ref, W1b_ref, w1c_ref, be1_ref, We2_ref, be2_ref,
               Wn1a_ref, Wn1b_ref, bn1_ref, Wn2_ref, bn2_ref,
               Wcx1_ref, bcx1_ref, Wcx2_ref, bcx2_ref,
               node_out_ref, coors_out_ref):
    feats_blk = feats_ref[0]
    coors_blk = coors_ref[0]
    dist_flat = dist_ref[0]

    P_i = (jnp.dot(feats_blk.astype(jnp.bfloat16), W1a_ref[...],
                   preferred_element_type=jnp.float32)
           + be1_ref[...]).astype(jnp.bfloat16)
    fj = g_ref[0, :, 0:64].astype(jnp.bfloat16)
    h = (_tile_k(P_i)
         + jnp.dot(fj, W1b_ref[...],
                   preferred_element_type=jnp.float32).astype(jnp.bfloat16)
         + dist_flat.astype(jnp.bfloat16) * w1c_ref[...])
    h = _silu(h)
    m_ij = _silu((jnp.dot(h, We2_ref[...],
                          preferred_element_type=jnp.float32)
                  + be2_ref[...]).astype(jnp.bfloat16))

    t12 = _silu((jnp.dot(m_ij, Wcx1_ref[...],
                         preferred_element_type=jnp.float32)
                 + bcx1_ref[...]).astype(jnp.bfloat16))
    cw2 = (jnp.dot(t12, Wcx2_ref[...],
                   preferred_element_type=jnp.float32)
           + bcx2_ref[...])
    cw = cw2[:, 0:1]
    cwx = cw2[:, 1:2]

    rel = _tile_k(coors_blk) - g_ref[0, :, 64:67]
    ai = _tile_k(aug_ref[0, :, 67:70])
    bj = g_ref[0, :, 67:70]
    cross = _roll3(ai, -1) * _roll3(bj, 1) - _roll3(ai, 1) * _roll3(bj, -1)
    contrib = cw * rel + cwx * cross

    pooled = jnp.dot(pool_ref[...],
                     jnp.concatenate([contrib.astype(jnp.bfloat16), m_ij],
                                     axis=1),
                     preferred_element_type=jnp.float32)
    csum = pooled[:, 0:3]
    m_i = pooled[:, 3:19]

    coors_out_ref[0] = csum + coors_blk

    nh = _silu((jnp.dot(feats_blk.astype(jnp.bfloat16), Wn1a_ref[...],
                        preferred_element_type=jnp.float32)
                + jnp.dot(m_i.astype(jnp.bfloat16), Wn1b_ref[...],
                          preferred_element_type=jnp.float32)
                + bn1_ref[...]).astype(jnp.bfloat16))
    node = (jnp.dot(nh, Wn2_ref[...],
                    preferred_element_type=jnp.float32)
            + bn2_ref[...] + feats_blk)
    node_out_ref[0] = node


@jax.jit
def kernel(feats, coors, W_e1, b_e1, W_e2, b_e2, W_n1, b_n1, W_n2, b_n2,
           W_c1, b_c1, W_c2, b_c2, W_x1, b_x1, W_x2, b_x2):
    b, n, d = feats.shape
    m_dim = W_e2.shape[1]
    nblk = n // R

    coors_t = jnp.transpose(coors, (0, 2, 1))

    pool = (jnp.arange(E, dtype=jnp.int32)[None, :] % R
            == jnp.arange(R, dtype=jnp.int32)[:, None]).astype(jnp.bfloat16)

    bf = jnp.bfloat16
    W1a = W_e1[:d].astype(bf)
    W1b = W_e1[d:2 * d].astype(bf)
    w1c = W_e1[2 * d:2 * d + 1].astype(bf)
    be1 = b_e1[None, :]
    We2 = W_e2.astype(bf)
    be2 = b_e2[None, :]
    Wn1a = W_n1[:d].astype(bf)
    Wn1b = W_n1[d:d + m_dim].astype(bf)
    bn1 = b_n1[None, :]
    Wn2 = W_n2.astype(bf)
    bn2 = b_n2[None, :]
    Wcx1 = jnp.concatenate([W_c1, W_x1], axis=1).astype(bf)
    bcx1 = jnp.concatenate([b_c1, b_x1])[None, :]
    zeros = jnp.zeros_like(W_c2)
    Wcx2 = jnp.concatenate(
        [jnp.concatenate([W_c2, zeros], axis=1),
         jnp.concatenate([zeros, W_x2], axis=1)], axis=0).astype(bf)
    bcx2 = jnp.concatenate([b_c2, b_x2])[None, :]

    full = lambda shp: pl.BlockSpec(shp, lambda s: (0,) * len(shp))
    step = lambda shp: pl.BlockSpec(shp, lambda s: (s,) + (0,) * (len(shp) - 1))
    fullA = lambda shp: pl.BlockSpec(shp, lambda bi, ii: (bi,) + (0,) * (len(shp) - 1))
    stepA = lambda shp: pl.BlockSpec(
        shp, lambda bi, ii: (bi * (n // R) + ii,) + (0,) * (len(shp) - 1))

    GROUP = 2
    SG = GROUP * nblk
    node_parts, coors_parts = [], []
    for g0 in range(0, b, GROUP):
        aug, idxg, distf = pl.pallas_call(
            _topk_block,
            grid=(GROUP, nblk),
            in_specs=[
                fullA((1, n, d)),
                fullA((1, n, 3)),
                fullA((1, 3, n)),
            ],
            out_specs=[
                stepA((1, R, TW)),
                stepA((1, K, R)),
                stepA((1, K, R)),
            ],
            out_shape=[
                jax.ShapeDtypeStruct((SG, R, TW), jnp.float32),
                jax.ShapeDtypeStruct((SG, K, R), jnp.int32),
                jax.ShapeDtypeStruct((SG, K, R), jnp.float32),
            ],
        )(feats[g0:g0 + GROUP], coors[g0:g0 + GROUP], coors_t[g0:g0 + GROUP])

        table = aug.reshape(GROUP * n, TW)
        indices = idxg.reshape(1, SG * E)
        distf = distf.reshape(SG, E, 1)
        gathered = _sc_gather(table, indices).reshape(SG, E, TW)

        node3, coors3 = pl.pallas_call(
            _mlp_block,
            grid=(SG,),
            in_specs=[
                step((1, E, TW)),
                step((1, E, 1)),
                step((1, R, TW)),
                step((1, R, d)),
                step((1, R, 3)),
                full(pool.shape),
                full(W1a.shape), full(W1b.shape), full(w1c.shape),
                full(be1.shape), full(We2.shape), full(be2.shape),
                full(Wn1a.shape), full(Wn1b.shape), full(bn1.shape),
                full(Wn2.shape), full(bn2.shape),
                full(Wcx1.shape), full(bcx1.shape), full(Wcx2.shape),
                full(bcx2.shape),
            ],
            out_specs=[
                step((1, R, d)),
                step((1, R, 3)),
            ],
            out_shape=[
                jax.ShapeDtypeStruct((SG, R, d), jnp.float32),
                jax.ShapeDtypeStruct((SG, R, 3), jnp.float32),
            ],
        )(gathered, distf, aug,
          feats[g0:g0 + GROUP].reshape(SG, R, d),
          coors[g0:g0 + GROUP].reshape(SG, R, 3), pool,
          W1a, W1b, w1c, be1, We2, be2, Wn1a, Wn1b, bn1, Wn2, bn2,
          Wcx1, bcx1, Wcx2, bcx2)
        node_parts.append(node3.reshape(GROUP, n, d))
        coors_parts.append(coors3.reshape(GROUP, n, 3))

    return (jnp.concatenate(node_parts, axis=0),
            jnp.concatenate(coors_parts, axis=0))


# speedup vs baseline: 1.3198x; 1.0158x over previous
import jax
import jax.numpy as jnp
from jax.experimental import pallas as pl
from jax.experimental.pallas import tpu as pltpu
from jax.experimental.pallas import tpu_sc as plsc

K = 32
R = 128
E = R * K
IDX_MASK = 1023
TW = 128
WINDOW = 128


def _silu(x):
    half = jnp.asarray(0.5, x.dtype)
    one = jnp.asarray(1.0, x.dtype)
    return half * x * (one + jnp.tanh(half * x))


def _roll3(x, shift):
    if shift == -1:
        return jnp.concatenate([x[:, 1:3], x[:, 0:1]], axis=1)
    return jnp.concatenate([x[:, 2:3], x[:, 0:2]], axis=1)


def _tile_k(x):
    return jnp.concatenate([x] * K, axis=0)


def _topk_block(feats_ref, coors_ref, coors_t_ref,
                aug_ref, idx_ref, dist_ref):
    n = feats_ref.shape[1]
    i0 = pl.program_id(1) * R

    coors_full = coors_ref[0]
    feats_blk = feats_ref[0, pl.ds(i0, R), :]
    coors_blk = coors_ref[0, pl.ds(i0, R), :]

    cxi = coors_t_ref[0, 0:1, pl.ds(i0, R)]
    cyi = coors_t_ref[0, 1:2, pl.ds(i0, R)]
    czi = coors_t_ref[0, 2:3, pl.ds(i0, R)]
    dx = coors_full[:, 0:1] - cxi
    dy = coors_full[:, 1:2] - cyi
    dz = coors_full[:, 2:3] - czi
    d2 = dx * dx + dy * dy + dz * dz

    bits = jax.lax.bitcast_convert_type(d2, jnp.int32)
    jrow = jax.lax.broadcasted_iota(jnp.int32, (n, R), 0)
    arr = (bits & jnp.int32(~IDX_MASK)) | jrow
    maxval = jnp.int32(0x7FFFFFFF)
    rows = []
    for _ in range(K):
        m = jnp.min(arr, axis=0, keepdims=True)
        rows.append(m)
        arr = jnp.where(arr == m, maxval, arr)
    packed_kr = jnp.concatenate(rows, axis=0)
    idx_ref[0] = (packed_kr & IDX_MASK) + pl.program_id(0) * n
    dist_ref[0] = jax.lax.bitcast_convert_type(
        packed_kr & jnp.int32(~IDX_MASK), jnp.float32)

    mean_c = jnp.mean(coors_full, axis=0, keepdims=True)
    cnm_blk = coors_blk - mean_c
    pad = jnp.zeros((R, TW - 70), jnp.float32)
    aug_ref[0] = jnp.concatenate(
        [feats_blk, coors_blk, cnm_blk, pad], axis=1)


def _sc_gather(table, indices):
    out_rows = indices.shape[1]
    width = table.shape[1]
    mesh = plsc.VectorSubcoreMesh(core_axis_name="core",
                                  subcore_axis_name="subcore")

    @pl.kernel(out_type=jax.ShapeDtypeStruct((out_rows, width), table.dtype),
               mesh=mesh)
    def gk(tab_hbm, idx_hbm, o_hbm):
        def body(i_vmem, o_vmem):
            pltpu.sync_copy(tab_hbm.at[i_vmem.at[0]], o_vmem)

        pltpu.emit_pipeline(
            body,
            grid=(out_rows // WINDOW,),
            in_specs=[pl.BlockSpec((1, WINDOW), index_map=lambda i: (0, i))],
            out_specs=[pl.BlockSpec((WINDOW, width),
                                    index_map=lambda i: (i, 0))],
            core_axis_name=("core", "subcore"),
            dimension_semantics=(pltpu.PARALLEL,),
        )(idx_hbm, o_hbm)

    return gk(table, indices)


def _mlp_block(g_ref, dist_ref, aug_ref, feats_ref, coors_ref, pool_ref,
               W1a_ref, W1b_ref, w1c_ref, be1_ref, We2_ref, be2_ref,
               Wn1a_ref, Wn1b_ref, bn1_ref, Wn2_ref, bn2_ref,
               Wcx1_ref, bcx1_ref, Wcx2_ref, bcx2_ref,
               node_out_ref, coors_out_ref):
    feats_blk = feats_ref[0]
    coors_blk = coors_ref[0]
    dist_flat = dist_ref[0]

    P_i = (jnp.dot(feats_blk.astype(jnp.bfloat16), W1a_ref[...],
                   preferred_element_type=jnp.float32)
           + be1_ref[...]).astype(jnp.bfloat16)
    fj = g_ref[0, :, 0:64].astype(jnp.bfloat16)
    h = (_tile_k(P_i)
         + jnp.dot(fj, W1b_ref[...],
                   preferred_element_type=jnp.float32).astype(jnp.bfloat16)
         + dist_flat.astype(jnp.bfloat16) * w1c_ref[...])
    h = _silu(h)
    m_ij = _silu((jnp.dot(h, We2_ref[...],
                          preferred_element_type=jnp.float32)
                  + be2_ref[...]).astype(jnp.bfloat16))

    t12 = _silu((jnp.dot(m_ij, Wcx1_ref[...],
                         preferred_element_type=jnp.float32)
                 + bcx1_ref[...]).astype(jnp.bfloat16))
    cw2 = (jnp.dot(t12, Wcx2_ref[...],
                   preferred_element_type=jnp.float32)
           + bcx2_ref[...])
    cw = cw2[:, 0:1]
    cwx = cw2[:, 1:2]

    rel = _tile_k(coors_blk) - g_ref[0, :, 64:67]
    ai = _tile_k(aug_ref[0, :, 67:70])
    bj = g_ref[0, :, 67:70]
    cross = _roll3(ai, -1) * _roll3(bj, 1) - _roll3(ai, 1) * _roll3(bj, -1)
    contrib = cw * rel + cwx * cross

    pooled = jnp.dot(pool_ref[...],
                     jnp.concatenate([contrib.astype(jnp.bfloat16), m_ij],
                                     axis=1),
                     preferred_element_type=jnp.float32)
    csum = pooled[:, 0:3]
    m_i = pooled[:, 3:19]

    coors_out_ref[0] = csum + coors_blk

    nh = _silu((jnp.dot(feats_blk.astype(jnp.bfloat16), Wn1a_ref[...],
                        preferred_element_type=jnp.float32)
                + jnp.dot(m_i.astype(jnp.bfloat16), Wn1b_ref[...],
                          preferred_element_type=jnp.float32)
                + bn1_ref[...]).astype(jnp.bfloat16))
    node = (jnp.dot(nh, Wn2_ref[...],
                    preferred_element_type=jnp.float32)
            + bn2_ref[...] + feats_blk)
    node_out_ref[0] = node


@jax.jit
def kernel(feats, coors, W_e1, b_e1, W_e2, b_e2, W_n1, b_n1, W_n2, b_n2,
           W_c1, b_c1, W_c2, b_c2, W_x1, b_x1, W_x2, b_x2):
    b, n, d = feats.shape
    m_dim = W_e2.shape[1]
    nblk = n // R

    coors_t = jnp.transpose(coors, (0, 2, 1))

    pool = (jnp.arange(E, dtype=jnp.int32)[None, :] % R
            == jnp.arange(R, dtype=jnp.int32)[:, None]).astype(jnp.bfloat16)

    bf = jnp.bfloat16
    W1a = W_e1[:d].astype(bf)
    W1b = W_e1[d:2 * d].astype(bf)
    w1c = W_e1[2 * d:2 * d + 1].astype(bf)
    be1 = b_e1[None, :]
    We2 = W_e2.astype(bf)
    be2 = b_e2[None, :]
    Wn1a = W_n1[:d].astype(bf)
    Wn1b = W_n1[d:d + m_dim].astype(bf)
    bn1 = b_n1[None, :]
    Wn2 = W_n2.astype(bf)
    bn2 = b_n2[None, :]
    Wcx1 = jnp.concatenate([W_c1, W_x1], axis=1).astype(bf)
    bcx1 = jnp.concatenate([b_c1, b_x1])[None, :]
    zeros = jnp.zeros_like(W_c2)
    Wcx2 = jnp.concatenate(
        [jnp.concatenate([W_c2, zeros], axis=1),
         jnp.concatenate([zeros, W_x2], axis=1)], axis=0).astype(bf)
    bcx2 = jnp.concatenate([b_c2, b_x2])[None, :]

    full = lambda shp: pl.BlockSpec(shp, lambda s: (0,) * len(shp))
    step = lambda shp: pl.BlockSpec(shp, lambda s: (s,) + (0,) * (len(shp) - 1))
    fullA = lambda shp: pl.BlockSpec(shp, lambda bi, ii: (bi,) + (0,) * (len(shp) - 1))
    stepA = lambda shp: pl.BlockSpec(
        shp, lambda bi, ii: (bi * (n // R) + ii,) + (0,) * (len(shp) - 1))

    GROUP = 4
    SG = GROUP * nblk
    node_parts, coors_parts = [], []
    for g0 in range(0, b, GROUP):
        aug, idxg, distf = pl.pallas_call(
            _topk_block,
            grid=(GROUP, nblk),
            in_specs=[
                fullA((1, n, d)),
                fullA((1, n, 3)),
                fullA((1, 3, n)),
            ],
            out_specs=[
                stepA((1, R, TW)),
                stepA((1, K, R)),
                stepA((1, K, R)),
            ],
            out_shape=[
                jax.ShapeDtypeStruct((SG, R, TW), jnp.float32),
                jax.ShapeDtypeStruct((SG, K, R), jnp.int32),
                jax.ShapeDtypeStruct((SG, K, R), jnp.float32),
            ],
        )(feats[g0:g0 + GROUP], coors[g0:g0 + GROUP], coors_t[g0:g0 + GROUP])

        table = aug.reshape(GROUP * n, TW)
        indices = idxg.reshape(1, SG * E)
        distf = distf.reshape(SG, E, 1)
        gathered = _sc_gather(table, indices).reshape(SG, E, TW)

        node3, coors3 = pl.pallas_call(
            _mlp_block,
            grid=(SG,),
            in_specs=[
                step((1, E, TW)),
                step((1, E, 1)),
                step((1, R, TW)),
                step((1, R, d)),
                step((1, R, 3)),
                full(pool.shape),
                full(W1a.shape), full(W1b.shape), full(w1c.shape),
                full(be1.shape), full(We2.shape), full(be2.shape),
                full(Wn1a.shape), full(Wn1b.shape), full(bn1.shape),
                full(Wn2.shape), full(bn2.shape),
                full(Wcx1.shape), full(bcx1.shape), full(Wcx2.shape),
                full(bcx2.shape),
            ],
            out_specs=[
                step((1, R, d)),
                step((1, R, 3)),
            ],
            out_shape=[
                jax.ShapeDtypeStruct((SG, R, d), jnp.float32),
                jax.ShapeDtypeStruct((SG, R, 3), jnp.float32),
            ],
        )(gathered, distf, aug,
          feats[g0:g0 + GROUP].reshape(SG, R, d),
          coors[g0:g0 + GROUP].reshape(SG, R, 3), pool,
          W1a, W1b, w1c, be1, We2, be2, Wn1a, Wn1b, bn1, Wn2, bn2,
          Wcx1, bcx1, Wcx2, bcx2)
        node_parts.append(node3.reshape(GROUP, n, d))
        coors_parts.append(coors3.reshape(GROUP, n, 3))

    return (jnp.concatenate(node_parts, axis=0),
            jnp.concatenate(coors_parts, axis=0))


# speedup vs baseline: 1.3545x; 1.0263x over previous
import jax
import jax.numpy as jnp
from jax.experimental import pallas as pl
from jax.experimental.pallas import tpu as pltpu
from jax.experimental.pallas import tpu_sc as plsc

K = 32
R = 256
E = R * K
IDX_MASK = 1023
TW = 128
WINDOW = 128


def _silu(x):
    half = jnp.asarray(0.5, x.dtype)
    one = jnp.asarray(1.0, x.dtype)
    return half * x * (one + jnp.tanh(half * x))


def _roll3(x, shift):
    if shift == -1:
        return jnp.concatenate([x[:, 1:3], x[:, 0:1]], axis=1)
    return jnp.concatenate([x[:, 2:3], x[:, 0:2]], axis=1)


def _tile_k(x):
    return jnp.concatenate([x] * K, axis=0)


def _topk_block(feats_ref, coors_ref, coors_t_ref,
                aug_ref, idx_ref, dist_ref):
    n = feats_ref.shape[1]
    i0 = pl.program_id(1) * R

    coors_full = coors_ref[0]
    feats_blk = feats_ref[0, pl.ds(i0, R), :]
    coors_blk = coors_ref[0, pl.ds(i0, R), :]

    cxi = coors_t_ref[0, 0:1, pl.ds(i0, R)]
    cyi = coors_t_ref[0, 1:2, pl.ds(i0, R)]
    czi = coors_t_ref[0, 2:3, pl.ds(i0, R)]
    dx = coors_full[:, 0:1] - cxi
    dy = coors_full[:, 1:2] - cyi
    dz = coors_full[:, 2:3] - czi
    d2 = dx * dx + dy * dy + dz * dz

    bits = jax.lax.bitcast_convert_type(d2, jnp.int32)
    jrow = jax.lax.broadcasted_iota(jnp.int32, (n, R), 0)
    arr = (bits & jnp.int32(~IDX_MASK)) | jrow
    maxval = jnp.int32(0x7FFFFFFF)
    rows = []
    for _ in range(K):
        m = jnp.min(arr, axis=0, keepdims=True)
        rows.append(m)
        arr = jnp.where(arr == m, maxval, arr)
    packed_kr = jnp.concatenate(rows, axis=0)
    idx_ref[0] = (packed_kr & IDX_MASK) + pl.program_id(0) * n
    dist_ref[0] = jax.lax.bitcast_convert_type(
        packed_kr & jnp.int32(~IDX_MASK), jnp.float32)

    mean_c = jnp.mean(coors_full, axis=0, keepdims=True)
    cnm_blk = coors_blk - mean_c
    pad = jnp.zeros((R, TW - 70), jnp.float32)
    aug_ref[0] = jnp.concatenate(
        [feats_blk, coors_blk, cnm_blk, pad], axis=1)


def _sc_gather(table, indices):
    out_rows = indices.shape[1]
    width = table.shape[1]
    mesh = plsc.VectorSubcoreMesh(core_axis_name="core",
                                  subcore_axis_name="subcore")

    @pl.kernel(out_type=jax.ShapeDtypeStruct((out_rows, width), table.dtype),
               mesh=mesh)
    def gk(tab_hbm, idx_hbm, o_hbm):
        def body(i_vmem, o_vmem):
            pltpu.sync_copy(tab_hbm.at[i_vmem.at[0]], o_vmem)

        pltpu.emit_pipeline(
            body,
            grid=(out_rows // WINDOW,),
            in_specs=[pl.BlockSpec((1, WINDOW), index_map=lambda i: (0, i))],
            out_specs=[pl.BlockSpec((WINDOW, width),
                                    index_map=lambda i: (i, 0))],
            core_axis_name=("core", "subcore"),
            dimension_semantics=(pltpu.PARALLEL,),
        )(idx_hbm, o_hbm)

    return gk(table, indices)


def _mlp_block(g_ref, dist_ref, aug_ref, feats_ref, coors_ref, pool_ref,
               W1a_ref, W1b_ref, w1c_ref, be1_ref, We2_ref, be2_ref,
               Wn1a_ref, Wn1b_ref, bn1_ref, Wn2_ref, bn2_ref,
               Wcx1_ref, bcx1_ref, Wcx2_ref, bcx2_ref,
               node_out_ref, coors_out_ref):
    feats_blk = feats_ref[0]
    coors_blk = coors_ref[0]
    dist_flat = dist_ref[0]

    P_i = (jnp.dot(feats_blk.astype(jnp.bfloat16), W1a_ref[...],
                   preferred_element_type=jnp.float32)
           + be1_ref[...]).astype(jnp.bfloat16)
    fj = g_ref[0, :, 0:64].astype(jnp.bfloat16)
    h = (_tile_k(P_i)
         + jnp.dot(fj, W1b_ref[...],
                   preferred_element_type=jnp.float32).astype(jnp.bfloat16)
         + dist_flat.astype(jnp.bfloat16) * w1c_ref[...])
    h = _silu(h)
    m_ij = _silu((jnp.dot(h, We2_ref[...],
                          preferred_element_type=jnp.float32)
                  + be2_ref[...]).astype(jnp.bfloat16))

    t12 = _silu((jnp.dot(m_ij, Wcx1_ref[...],
                         preferred_element_type=jnp.float32)
                 + bcx1_ref[...]).astype(jnp.bfloat16))
    cw2 = (jnp.dot(t12, Wcx2_ref[...],
                   preferred_element_type=jnp.float32)
           + bcx2_ref[...])
    cw = cw2[:, 0:1]
    cwx = cw2[:, 1:2]

    rel = _tile_k(coors_blk) - g_ref[0, :, 64:67]
    ai = _tile_k(aug_ref[0, :, 67:70])
    bj = g_ref[0, :, 67:70]
    cross = _roll3(ai, -1) * _roll3(bj, 1) - _roll3(ai, 1) * _roll3(bj, -1)
    contrib = cw * rel + cwx * cross

    pooled = jnp.dot(pool_ref[...],
                     jnp.concatenate([contrib.astype(jnp.bfloat16), m_ij],
                                     axis=1),
                     preferred_element_type=jnp.float32)
    csum = pooled[:, 0:3]
    m_i = pooled[:, 3:19]

    coors_out_ref[0] = csum + coors_blk

    nh = _silu((jnp.dot(feats_blk.astype(jnp.bfloat16), Wn1a_ref[...],
                        preferred_element_type=jnp.float32)
                + jnp.dot(m_i.astype(jnp.bfloat16), Wn1b_ref[...],
                          preferred_element_type=jnp.float32)
                + bn1_ref[...]).astype(jnp.bfloat16))
    node = (jnp.dot(nh, Wn2_ref[...],
                    preferred_element_type=jnp.float32)
            + bn2_ref[...] + feats_blk)
    node_out_ref[0] = node


@jax.jit
def kernel(feats, coors, W_e1, b_e1, W_e2, b_e2, W_n1, b_n1, W_n2, b_n2,
           W_c1, b_c1, W_c2, b_c2, W_x1, b_x1, W_x2, b_x2):
    b, n, d = feats.shape
    m_dim = W_e2.shape[1]
    nblk = n // R

    coors_t = jnp.transpose(coors, (0, 2, 1))

    pool = (jnp.arange(E, dtype=jnp.int32)[None, :] % R
            == jnp.arange(R, dtype=jnp.int32)[:, None]).astype(jnp.bfloat16)

    bf = jnp.bfloat16
    W1a = W_e1[:d].astype(bf)
    W1b = W_e1[d:2 * d].astype(bf)
    w1c = W_e1[2 * d:2 * d + 1].astype(bf)
    be1 = b_e1[None, :]
    We2 = W_e2.astype(bf)
    be2 = b_e2[None, :]
    Wn1a = W_n1[:d].astype(bf)
    Wn1b = W_n1[d:d + m_dim].astype(bf)
    bn1 = b_n1[None, :]
    Wn2 = W_n2.astype(bf)
    bn2 = b_n2[None, :]
    Wcx1 = jnp.concatenate([W_c1, W_x1], axis=1).astype(bf)
    bcx1 = jnp.concatenate([b_c1, b_x1])[None, :]
    zeros = jnp.zeros_like(W_c2)
    Wcx2 = jnp.concatenate(
        [jnp.concatenate([W_c2, zeros], axis=1),
         jnp.concatenate([zeros, W_x2], axis=1)], axis=0).astype(bf)
    bcx2 = jnp.concatenate([b_c2, b_x2])[None, :]

    full = lambda shp: pl.BlockSpec(shp, lambda s: (0,) * len(shp))
    step = lambda shp: pl.BlockSpec(shp, lambda s: (s,) + (0,) * (len(shp) - 1))
    fullA = lambda shp: pl.BlockSpec(shp, lambda bi, ii: (bi,) + (0,) * (len(shp) - 1))
    stepA = lambda shp: pl.BlockSpec(
        shp, lambda bi, ii: (bi * (n // R) + ii,) + (0,) * (len(shp) - 1))

    GROUP = 4
    SG = GROUP * nblk
    node_parts, coors_parts = [], []
    for g0 in range(0, b, GROUP):
        aug, idxg, distf = pl.pallas_call(
            _topk_block,
            grid=(GROUP, nblk),
            in_specs=[
                fullA((1, n, d)),
                fullA((1, n, 3)),
                fullA((1, 3, n)),
            ],
            out_specs=[
                stepA((1, R, TW)),
                stepA((1, K, R)),
                stepA((1, K, R)),
            ],
            out_shape=[
                jax.ShapeDtypeStruct((SG, R, TW), jnp.float32),
                jax.ShapeDtypeStruct((SG, K, R), jnp.int32),
                jax.ShapeDtypeStruct((SG, K, R), jnp.float32),
            ],
        )(feats[g0:g0 + GROUP], coors[g0:g0 + GROUP], coors_t[g0:g0 + GROUP])

        table = aug.reshape(GROUP * n, TW)
        indices = idxg.reshape(1, SG * E)
        distf = distf.reshape(SG, E, 1)
        gathered = _sc_gather(table, indices).reshape(SG, E, TW)

        node3, coors3 = pl.pallas_call(
            _mlp_block,
            grid=(SG,),
            in_specs=[
                step((1, E, TW)),
                step((1, E, 1)),
                step((1, R, TW)),
                step((1, R, d)),
                step((1, R, 3)),
                full(pool.shape),
                full(W1a.shape), full(W1b.shape), full(w1c.shape),
                full(be1.shape), full(We2.shape), full(be2.shape),
                full(Wn1a.shape), full(Wn1b.shape), full(bn1.shape),
                full(Wn2.shape), full(bn2.shape),
                full(Wcx1.shape), full(bcx1.shape), full(Wcx2.shape),
                full(bcx2.shape),
            ],
            out_specs=[
                step((1, R, d)),
                step((1, R, 3)),
            ],
            out_shape=[
                jax.ShapeDtypeStruct((SG, R, d), jnp.float32),
                jax.ShapeDtypeStruct((SG, R, 3), jnp.float32),
            ],
        )(gathered, distf, aug,
          feats[g0:g0 + GROUP].reshape(SG, R, d),
          coors[g0:g0 + GROUP].reshape(SG, R, 3), pool,
          W1a, W1b, w1c, be1, We2, be2, Wn1a, Wn1b, bn1, Wn2, bn2,
          Wcx1, bcx1, Wcx2, bcx2)
        node_parts.append(node3.reshape(GROUP, n, d))
        coors_parts.append(coors3.reshape(GROUP, n, 3))

    return (jnp.concatenate(node_parts, axis=0),
            jnp.concatenate(coors_parts, axis=0))
